# speedup vs baseline: 18.7377x; 18.7377x over previous
import functools

import jax
import jax.numpy as jnp
from jax import lax
from jax.experimental import pallas as pl
from jax.experimental.pallas import tpu as pltpu

_K = 3
_PAD = 1
_OG = 2
_K2 = _K * _K


def _dc_kernel(xt_ref, wcat_ref, bcat_ref, wm_ref, out_ref, patch_ref, val_ref,
               *, H, W, C, Cg):
  h = pl.program_id(1)

  lane = lax.broadcasted_iota(jnp.int32, (1, W), 1)
  lane_c = lax.broadcasted_iota(jnp.int32, (C, W), 1)

  for ki in range(_K):
    row = h - _PAD + ki
    rowc = jnp.clip(row, 0, H - 1)
    slab = xt_ref[0, rowc, :, :]
    valid = jnp.logical_and(row >= 0, row < H)
    slab = jnp.where(valid, slab, 0.0)
    for kj in range(_K):
      sh = kj - _PAD
      if sh == 0:
        shifted = slab
      elif sh < 0:
        shifted = pltpu.roll(slab, -sh, axis=1)
        shifted = jnp.where(lane_c < -sh, 0.0, shifted)
      else:
        shifted = pltpu.roll(slab, W - sh, axis=1)
        shifted = jnp.where(lane_c >= W - sh, 0.0, shifted)
      r0 = (ki * _K + kj) * C
      patch_ref[r0:r0 + C, :] = shifted

  om = jnp.dot(wcat_ref[...], patch_ref[...],
               preferred_element_type=jnp.float32) + bcat_ref[...]

  hf = (h - _PAD).astype(jnp.float32)
  for g in range(_OG):
    gs = g * Cg
    for k in range(_K2):
      ki, kj = k // _K, k % _K
      orow = (g * _K2 + k) * 2
      dy = om[orow:orow + 1, :]
      dx = om[orow + 1:orow + 2, :]
      mr = 2 * _OG * _K2 + g * _K2 + k
      mrow = jax.nn.sigmoid(om[mr:mr + 1, :])

      py = dy + (hf + ki)
      px = dx + (lane - _PAD + kj).astype(jnp.float32)
      y0f = jnp.floor(py)
      x0f = jnp.floor(px)
      wy = py - y0f
      wx = px - x0f
      y0 = y0f.astype(jnp.int32)
      x0 = x0f.astype(jnp.int32)
      x1 = x0 + 1

      x0c = jnp.clip(x0, 0, W - 1)
      x1c = jnp.clip(x1, 0, W - 1)
      vx0 = jnp.where(jnp.logical_and(x0 >= 0, x0 <= W - 1), 1.0, 0.0)
      vx1 = jnp.where(jnp.logical_and(x1 >= 0, x1 <= W - 1), 1.0, 0.0)
      wxl = (1.0 - wx) * vx0 * mrow
      wxr = wx * vx1 * mrow

      x0cb = jnp.broadcast_to(x0c, (Cg, W))
      x1cb = jnp.broadcast_to(x1c, (Cg, W))
      wxlb = jnp.broadcast_to(wxl, (Cg, W))
      wxrb = jnp.broadcast_to(wxr, (Cg, W))
      cy0 = 1.0 - wy
      cy1 = wy

      lo = jnp.clip(jnp.min(y0), 0, H - 1)
      hi = jnp.clip(jnp.max(y0) + 1, 0, H - 1)

      def body(r, acc, *, gs=gs, x0cb=x0cb, x1cb=x1cb, wxlb=wxlb, wxrb=wxrb,
               y0=y0, cy0=cy0, cy1=cy1):
        slab = xt_ref[0, r, gs:gs + Cg, :]
        t0 = jnp.take_along_axis(slab, x0cb, axis=1)
        t1 = jnp.take_along_axis(slab, x1cb, axis=1)
        hx = t0 * wxlb + t1 * wxrb
        cy = jnp.where(y0 == r, cy0, 0.0) + jnp.where(y0 == r - 1, cy1, 0.0)
        return acc + hx * jnp.broadcast_to(cy, (Cg, W))

      acc = lax.fori_loop(lo, hi + 1, body, jnp.zeros((Cg, W), jnp.float32))
      v0 = (g * _K2 + k) * Cg
      val_ref[v0:v0 + Cg, :] = acc

  out_ref[0, 0, :, :] = jnp.dot(wm_ref[...], val_ref[...],
                                preferred_element_type=jnp.float32)


@jax.jit
def kernel(x, w_main, w_off, b_off, w_mask, b_mask):
  B, C, H, W = x.shape
  O = w_main.shape[0]
  Cg = C // _OG
  n_off = 2 * _OG * _K2
  n_cat = n_off + _OG * _K2
  n_pad = 56

  xt = jnp.transpose(x, (0, 2, 1, 3))

  wcat = jnp.concatenate([w_off, w_mask], axis=0)
  wcat = wcat.transpose(0, 2, 3, 1).reshape(n_cat, _K2 * C)
  wcat = jnp.pad(wcat, ((0, n_pad - n_cat), (0, 0)))
  bcat = jnp.concatenate([b_off, b_mask], axis=0)
  bcat = jnp.pad(bcat, (0, n_pad - n_cat))
  bcat = jnp.broadcast_to(bcat[:, None], (n_pad, W))

  wm = w_main.reshape(O, _OG, Cg, _K, _K)
  wm = wm.transpose(0, 1, 3, 4, 2).reshape(O, _OG * _K2 * Cg)

  body = functools.partial(_dc_kernel, H=H, W=W, C=C, Cg=Cg)
  out_t = pl.pallas_call(
      body,
      grid=(B, H),
      in_specs=[
          pl.BlockSpec((1, H, C, W), lambda b, h: (b, 0, 0, 0)),
          pl.BlockSpec((n_pad, _K2 * C), lambda b, h: (0, 0)),
          pl.BlockSpec((n_pad, W), lambda b, h: (0, 0)),
          pl.BlockSpec((O, _OG * _K2 * Cg), lambda b, h: (0, 0)),
      ],
      out_specs=pl.BlockSpec((1, 1, O, W), lambda b, h: (b, h, 0, 0)),
      out_shape=jax.ShapeDtypeStruct((B, H, O, W), jnp.float32),
      scratch_shapes=[
          pltpu.VMEM((_K2 * C, W), jnp.float32),
          pltpu.VMEM((_OG * _K2 * Cg, W), jnp.float32),
      ],
      compiler_params=pltpu.CompilerParams(
          dimension_semantics=(pltpu.GridDimensionSemantics.PARALLEL,
                               pltpu.GridDimensionSemantics.ARBITRARY),
          vmem_limit_bytes=64 * 1024 * 1024,
      ),
  )(xt, wcat, bcat, wm)

  return jnp.transpose(out_t, (0, 2, 1, 3))


# speedup vs baseline: 42.3543x; 2.2604x over previous
import functools

import jax
import jax.numpy as jnp
from jax import lax
from jax.experimental import pallas as pl
from jax.experimental.pallas import tpu as pltpu

_K = 3
_PAD = 1
_OG = 2
_K2 = _K * _K
_NT = _OG * _K2
_WIN = 5


def _dc_kernel(xt_ref, wcat_ref, bcat_ref, wm_ref, out_ref, patch_ref, val_ref,
               *, H, W, C, Cg):
  h = pl.program_id(1)

  lane = lax.broadcasted_iota(jnp.int32, (1, W), 1)
  lane_c = lax.broadcasted_iota(jnp.int32, (C, W), 1)

  for ki in range(_K):
    row = h - _PAD + ki
    rowc = jnp.clip(row, 0, H - 1)
    slab = xt_ref[0, rowc, :, :]
    valid = jnp.logical_and(row >= 0, row < H)
    slab = jnp.where(valid, slab, 0.0)
    for kj in range(_K):
      sh = kj - _PAD
      if sh == 0:
        shifted = slab
      elif sh < 0:
        shifted = pltpu.roll(slab, -sh, axis=1)
        shifted = jnp.where(lane_c < -sh, 0.0, shifted)
      else:
        shifted = pltpu.roll(slab, W - sh, axis=1)
        shifted = jnp.where(lane_c >= W - sh, 0.0, shifted)
      r0 = (ki * _K + kj) * C
      patch_ref[r0:r0 + C, :] = shifted

  om = jnp.dot(wcat_ref[...], patch_ref[...],
               preferred_element_type=jnp.float32) + bcat_ref[...]

  hf = (h - _PAD).astype(jnp.float32)
  dy_all = om[0:_NT, :]
  dx_all = om[_NT:2 * _NT, :]
  m_all = jax.nn.sigmoid(om[2 * _NT:3 * _NT, :])

  si = lax.broadcasted_iota(jnp.int32, (_NT, W), 0)
  kiv = ((si % _K2) // _K).astype(jnp.float32)
  kjv = (si % _K).astype(jnp.float32)
  lanef = jnp.broadcast_to(lane, (_NT, W)).astype(jnp.float32)

  py = dy_all + (hf + kiv)
  px = dx_all + (lanef - _PAD) + kjv
  y0f = jnp.floor(py)
  x0f = jnp.floor(px)
  wy = py - y0f
  wx = px - x0f
  x0 = x0f.astype(jnp.int32)
  x1 = x0 + 1
  x0c = jnp.clip(x0, 0, W - 1)
  x1c = jnp.clip(x1, 0, W - 1)
  vx0 = jnp.where(jnp.logical_and(x0 >= 0, x0 <= W - 1), 1.0, 0.0)
  vx1 = jnp.where(jnp.logical_and(x1 >= 0, x1 <= W - 1), 1.0, 0.0)
  mwxl = (1.0 - wx) * vx0 * m_all
  mwxr = wx * vx1 * m_all
  cy0a = 1.0 - wy
  cy1a = wy
  ymin = jnp.min(y0f, axis=1, keepdims=True)
  ymax = jnp.max(y0f, axis=1, keepdims=True)

  for g in range(_OG):
    gs = g * Cg
    for k in range(_K2):
      i = g * _K2 + k
      y0f_i = y0f[i:i + 1, :]
      cy0_i = cy0a[i:i + 1, :]
      cy1_i = cy1a[i:i + 1, :]
      mwxl_i = jnp.broadcast_to(mwxl[i:i + 1, :], (Cg, W))
      mwxr_i = jnp.broadcast_to(mwxr[i:i + 1, :], (Cg, W))
      x0cb = jnp.broadcast_to(x0c[i:i + 1, :], (Cg, W))
      x1cb = jnp.broadcast_to(x1c[i:i + 1, :], (Cg, W))

      lo_f = jnp.clip(ymin[i, 0], 0.0, float(H - 1))
      hi_f = jnp.clip(ymax[i, 0] + 1.0, 0.0, float(H - 1))
      lo = lo_f.astype(jnp.int32)
      hi = hi_f.astype(jnp.int32)

      def contrib(r, rf, slab, *, x0cb=x0cb, x1cb=x1cb, mwxl_i=mwxl_i,
                  mwxr_i=mwxr_i, y0f_i=y0f_i, cy0_i=cy0_i, cy1_i=cy1_i):
        t0 = jnp.take_along_axis(slab, x0cb, axis=1)
        t1 = jnp.take_along_axis(slab, x1cb, axis=1)
        cy = (jnp.where(y0f_i == rf, cy0_i, 0.0)
              + jnp.where(y0f_i == rf - 1.0, cy1_i, 0.0))
        a = jnp.broadcast_to(cy, (Cg, W))
        return t0 * (mwxl_i * a) + t1 * (mwxr_i * a)

      acc = jnp.zeros((Cg, W), jnp.float32)
      for u in range(_WIN):
        r = lo + u
        rc = jnp.minimum(r, H - 1)
        slab = xt_ref[0, rc, gs:gs + Cg, :]
        c = contrib(r, r.astype(jnp.float32), slab)
        c = jnp.where(r <= H - 1, c, 0.0)
        acc = acc + c

      def body(r, acc, *, gs=gs, contrib=contrib):
        slab = xt_ref[0, r, gs:gs + Cg, :]
        return acc + contrib(r, r.astype(jnp.float32), slab)

      acc = lax.fori_loop(lo + _WIN, hi + 1, body, acc)
      v0 = i * Cg
      val_ref[v0:v0 + Cg, :] = acc

  out_ref[0, 0, :, :] = jnp.dot(wm_ref[...], val_ref[...],
                                preferred_element_type=jnp.float32)


@jax.jit
def kernel(x, w_main, w_off, b_off, w_mask, b_mask):
  B, C, H, W = x.shape
  O = w_main.shape[0]
  Cg = C // _OG
  n_cat = 3 * _NT
  n_pad = 56

  xt = jnp.transpose(x, (0, 2, 1, 3))

  w_off_r = w_off.reshape(_NT, 2, C, _K, _K)
  b_off_r = b_off.reshape(_NT, 2)
  wcat = jnp.concatenate([w_off_r[:, 0], w_off_r[:, 1], w_mask], axis=0)
  wcat = wcat.transpose(0, 2, 3, 1).reshape(n_cat, _K2 * C)
  wcat = jnp.pad(wcat, ((0, n_pad - n_cat), (0, 0)))
  bcat = jnp.concatenate([b_off_r[:, 0], b_off_r[:, 1], b_mask], axis=0)
  bcat = jnp.pad(bcat, (0, n_pad - n_cat))
  bcat = jnp.broadcast_to(bcat[:, None], (n_pad, W))

  wm = w_main.reshape(O, _OG, Cg, _K, _K)
  wm = wm.transpose(0, 1, 3, 4, 2).reshape(O, _NT * Cg)

  body = functools.partial(_dc_kernel, H=H, W=W, C=C, Cg=Cg)
  out_t = pl.pallas_call(
      body,
      grid=(B, H),
      in_specs=[
          pl.BlockSpec((1, H, C, W), lambda b, h: (b, 0, 0, 0)),
          pl.BlockSpec((n_pad, _K2 * C), lambda b, h: (0, 0)),
          pl.BlockSpec((n_pad, W), lambda b, h: (0, 0)),
          pl.BlockSpec((O, _NT * Cg), lambda b, h: (0, 0)),
      ],
      out_specs=pl.BlockSpec((1, 1, O, W), lambda b, h: (b, h, 0, 0)),
      out_shape=jax.ShapeDtypeStruct((B, H, O, W), jnp.float32),
      scratch_shapes=[
          pltpu.VMEM((_K2 * C, W), jnp.float32),
          pltpu.VMEM((_NT * Cg, W), jnp.float32),
      ],
      compiler_params=pltpu.CompilerParams(
          dimension_semantics=(pltpu.GridDimensionSemantics.PARALLEL,
                               pltpu.GridDimensionSemantics.ARBITRARY),
          vmem_limit_bytes=64 * 1024 * 1024,
      ),
  )(xt, wcat, bcat, wm)

  return jnp.transpose(out_t, (0, 2, 1, 3))


# speedup vs baseline: 90.3390x; 2.1329x over previous
import functools

import jax
import jax.numpy as jnp
from jax import lax
from jax.experimental import pallas as pl
from jax.experimental.pallas import tpu as pltpu

_K = 3
_PAD = 1
_OG = 2
_K2 = _K * _K
_NT = _OG * _K2
_WIN = 4


def _dc_kernel(xt_ref, wcat_ref, bcat_ref, wm_ref, out_ref, patch_ref, val_ref,
               *, H, W, C, Cg):
  h = pl.program_id(1)

  lane = lax.broadcasted_iota(jnp.int32, (1, W), 1)
  lane_c = lax.broadcasted_iota(jnp.int32, (C, W), 1)

  for ki in range(_K):
    row = h - _PAD + ki
    rowc = jnp.clip(row, 0, H - 1)
    slab = xt_ref[0, rowc, :, :]
    valid = jnp.logical_and(row >= 0, row < H)
    slab = jnp.where(valid, slab, 0.0)
    for kj in range(_K):
      sh = kj - _PAD
      if sh == 0:
        shifted = slab
      elif sh < 0:
        shifted = pltpu.roll(slab, -sh, axis=1)
        shifted = jnp.where(lane_c < -sh, 0.0, shifted)
      else:
        shifted = pltpu.roll(slab, W - sh, axis=1)
        shifted = jnp.where(lane_c >= W - sh, 0.0, shifted)
      r0 = (ki * _K + kj) * C
      patch_ref[r0:r0 + C, :] = shifted

  om = jnp.dot(wcat_ref[...], patch_ref[...],
               preferred_element_type=jnp.float32) + bcat_ref[...]

  hf = (h - _PAD).astype(jnp.float32)
  dy_all = om[0:_NT, :]
  dx_all = om[_NT:2 * _NT, :]
  m_all = jax.nn.sigmoid(om[2 * _NT:3 * _NT, :])

  si = lax.broadcasted_iota(jnp.int32, (_NT, W), 0)
  kiv = ((si % _K2) // _K).astype(jnp.float32)
  kjv = (si % _K).astype(jnp.float32)
  lanef = jnp.broadcast_to(lane, (_NT, W)).astype(jnp.float32)

  py = dy_all + (hf + kiv)
  px = dx_all + (lanef - _PAD) + kjv
  y0f = jnp.floor(py)
  x0f = jnp.floor(px)
  wy = py - y0f
  wx = px - x0f
  x0 = x0f.astype(jnp.int32)
  x1 = x0 + 1
  x0c = jnp.clip(x0, 0, W - 1)
  x1c = jnp.clip(x1, 0, W - 1)
  vx0 = jnp.where(jnp.logical_and(x0 >= 0, x0 <= W - 1), 1.0, 0.0)
  vx1 = jnp.where(jnp.logical_and(x1 >= 0, x1 <= W - 1), 1.0, 0.0)
  mwxl = (1.0 - wx) * vx0 * m_all
  mwxr = wx * vx1 * m_all
  cy0a = 1.0 - wy
  cy1a = wy
  ymin = jnp.min(y0f, axis=1, keepdims=True)
  ymax = jnp.max(y0f, axis=1, keepdims=True)
  lo_fa = jnp.clip(ymin, 0.0, float(H - 1))
  hi_fa = jnp.clip(ymax + 1.0, 0.0, float(H - 1))
  span = jnp.max(hi_fa - lo_fa)

  def contrib(rf, slab, i, x0cb, x1cb):
    t0 = jnp.take_along_axis(slab, x0cb, axis=1)
    t1 = jnp.take_along_axis(slab, x1cb, axis=1)
    cy = (jnp.where(y0f[i:i + 1, :] == rf, cy0a[i:i + 1, :], 0.0)
          + jnp.where(y0f[i:i + 1, :] == rf - 1.0, cy1a[i:i + 1, :], 0.0))
    la = jnp.broadcast_to(mwxl[i:i + 1, :] * cy, (Cg, W))
    ra = jnp.broadcast_to(mwxr[i:i + 1, :] * cy, (Cg, W))
    return t0 * la + t1 * ra

  for g in range(_OG):
    gs = g * Cg
    for k in range(_K2):
      i = g * _K2 + k
      x0cb = jnp.broadcast_to(x0c[i:i + 1, :], (Cg, W))
      x1cb = jnp.broadcast_to(x1c[i:i + 1, :], (Cg, W))
      lo = lo_fa[i, 0].astype(jnp.int32)

      acc = jnp.zeros((Cg, W), jnp.float32)
      for u in range(_WIN):
        r = lo + u
        rc = jnp.minimum(r, H - 1)
        slab = xt_ref[0, rc, gs:gs + Cg, :]
        rf = jnp.where(r <= H - 1, r.astype(jnp.float32), jnp.nan)
        acc = acc + contrib(rf, slab, i, x0cb, x1cb)

      val_ref[i * Cg:(i + 1) * Cg, :] = acc

  @pl.when(span > float(_WIN) - 0.5)
  def _residual():
    for g in range(_OG):
      gs = g * Cg
      for k in range(_K2):
        i = g * _K2 + k
        x0cb = jnp.broadcast_to(x0c[i:i + 1, :], (Cg, W))
        x1cb = jnp.broadcast_to(x1c[i:i + 1, :], (Cg, W))
        lo = lo_fa[i, 0].astype(jnp.int32)
        hi = hi_fa[i, 0].astype(jnp.int32)

        def body(r, acc, *, gs=gs, i=i, x0cb=x0cb, x1cb=x1cb):
          slab = xt_ref[0, r, gs:gs + Cg, :]
          return acc + contrib(r.astype(jnp.float32), slab, i, x0cb, x1cb)

        acc = lax.fori_loop(lo + _WIN, hi + 1, body,
                            jnp.zeros((Cg, W), jnp.float32))
        val_ref[i * Cg:(i + 1) * Cg, :] = val_ref[i * Cg:(i + 1) * Cg, :] + acc

  out_ref[0, 0, :, :] = jnp.dot(wm_ref[...], val_ref[...],
                                preferred_element_type=jnp.float32)


@jax.jit
def kernel(x, w_main, w_off, b_off, w_mask, b_mask):
  B, C, H, W = x.shape
  O = w_main.shape[0]
  Cg = C // _OG
  n_cat = 3 * _NT
  n_pad = 56

  xt = jnp.transpose(x, (0, 2, 1, 3))

  w_off_r = w_off.reshape(_NT, 2, C, _K, _K)
  b_off_r = b_off.reshape(_NT, 2)
  wcat = jnp.concatenate([w_off_r[:, 0], w_off_r[:, 1], w_mask], axis=0)
  wcat = wcat.transpose(0, 2, 3, 1).reshape(n_cat, _K2 * C)
  wcat = jnp.pad(wcat, ((0, n_pad - n_cat), (0, 0)))
  bcat = jnp.concatenate([b_off_r[:, 0], b_off_r[:, 1], b_mask], axis=0)
  bcat = jnp.pad(bcat, (0, n_pad - n_cat))
  bcat = jnp.broadcast_to(bcat[:, None], (n_pad, W))

  wm = w_main.reshape(O, _OG, Cg, _K, _K)
  wm = wm.transpose(0, 1, 3, 4, 2).reshape(O, _NT * Cg)

  body = functools.partial(_dc_kernel, H=H, W=W, C=C, Cg=Cg)
  out_t = pl.pallas_call(
      body,
      grid=(B, H),
      in_specs=[
          pl.BlockSpec((1, H, C, W), lambda b, h: (b, 0, 0, 0)),
          pl.BlockSpec((n_pad, _K2 * C), lambda b, h: (0, 0)),
          pl.BlockSpec((n_pad, W), lambda b, h: (0, 0)),
          pl.BlockSpec((O, _NT * Cg), lambda b, h: (0, 0)),
      ],
      out_specs=pl.BlockSpec((1, 1, O, W), lambda b, h: (b, h, 0, 0)),
      out_shape=jax.ShapeDtypeStruct((B, H, O, W), jnp.float32),
      scratch_shapes=[
          pltpu.VMEM((_K2 * C, W), jnp.float32),
          pltpu.VMEM((_NT * Cg, W), jnp.float32),
      ],
      compiler_params=pltpu.CompilerParams(
          dimension_semantics=(pltpu.GridDimensionSemantics.PARALLEL,
                               pltpu.GridDimensionSemantics.ARBITRARY),
          vmem_limit_bytes=64 * 1024 * 1024,
      ),
  )(xt, wcat, bcat, wm)

  return jnp.transpose(out_t, (0, 2, 1, 3))
